# speedup vs baseline: 2.4901x; 2.4901x over previous
import functools

import jax
import jax.numpy as jnp
from jax import lax
from jax.experimental import pallas as pl
from jax.experimental.pallas import tpu as pltpu
from jax.experimental.pallas import tpu_sc as plsc

_D = 768
_NC = 2
_NS = 16
_NW = _NC * _NS
_B = 4 * 8192
_BPW = _B // _NW
_CHUNK = 64
_NCHUNK = _BPW // _CHUNK


def _make_gather():
    mesh = plsc.VectorSubcoreMesh(core_axis_name="c", subcore_axis_name="s")

    @functools.partial(
        pl.kernel,
        mesh=mesh,
        out_type=jax.ShapeDtypeStruct((_B, _D), jnp.float32),
        scratch_types=[
            pltpu.VMEM((_BPW,), jnp.int32),
            pltpu.VMEM((_CHUNK, _D), jnp.float32),
            pltpu.VMEM((_CHUNK, _D), jnp.float32),
            pltpu.SemaphoreType.DMA,
        ],
    )
    def gather_kernel(table_hbm, idx_hbm, out_hbm, idx_v, buf0, buf1, sem_g):
        wid = lax.axis_index("s") * _NC + lax.axis_index("c")
        base = wid * _BPW
        pltpu.sync_copy(idx_hbm.at[pl.ds(base, _BPW)], idx_v)
        bufs = (buf0, buf1)
        copies = [
            pltpu.async_copy(
                table_hbm.at[idx_v.at[pl.ds(0, _CHUNK)]], buf0, sem_g
            )
        ]
        for g in range(_NCHUNK):
            if g + 1 < _NCHUNK:
                copies.append(
                    pltpu.async_copy(
                        table_hbm.at[idx_v.at[pl.ds((g + 1) * _CHUNK, _CHUNK)]],
                        bufs[(g + 1) % 2],
                        sem_g,
                    )
                )
            copies[g].wait()
            pltpu.sync_copy(
                bufs[g % 2], out_hbm.at[pl.ds(base + g * _CHUNK, _CHUNK)]
            )

    return gather_kernel


_gather = _make_gather()


def kernel(input_ids, position_ids, pos_table):
    del input_ids
    flat_ids = position_ids.reshape(-1)
    out = _gather(pos_table, flat_ids)
    return out.reshape(position_ids.shape + (pos_table.shape[1],))
